# baseline (speedup 1.0000x reference)
import functools

import jax
import jax.numpy as jnp
from jax import lax
from jax.experimental import pallas as pl
from jax.experimental.pallas import tpu as pltpu
from jax.experimental.pallas import tpu_sc as plsc

D_VOCAB = 1000000
D_EMB = 32
SCALE = float(D_EMB) ** 0.5

_NC = 2
_NS = 16
_NW = _NC * _NS

_B = 4096 * 200
_B_PER_W = _B // _NW
_CHUNK = 160
_N_CHUNKS = _B_PER_W // _CHUNK
_DEPTH = 4

_mesh = plsc.VectorSubcoreMesh(core_axis_name="c", subcore_axis_name="s")


@functools.partial(
    pl.kernel,
    mesh=_mesh,
    compiler_params=pltpu.CompilerParams(use_tc_tiling_on_sc=False),
    out_type=jax.ShapeDtypeStruct((_B, 128), jnp.float32),
    scratch_types=(
        [pltpu.VMEM((_B_PER_W,), jnp.int32)]
        + [pltpu.VMEM((_CHUNK, 16), jnp.float32) for _ in range(2 * _DEPTH)]
        + [pltpu.VMEM((_CHUNK, 128), jnp.float32) for _ in range(2)]
        + [pltpu.SemaphoreType.DMA for _ in range(2 * _DEPTH + 2)]
    ),
)
def _embed_sc(idx_hbm, tl_hbm, tr_hbm, out_hbm, idx_v, *bufs_and_sems):
    rows_l = bufs_and_sems[:_DEPTH]
    rows_r = bufs_and_sems[_DEPTH:2 * _DEPTH]
    wide = bufs_and_sems[2 * _DEPTH:2 * _DEPTH + 2]
    gsem_l = bufs_and_sems[2 * _DEPTH + 2:3 * _DEPTH + 2]
    gsem_r = bufs_and_sems[3 * _DEPTH + 2:4 * _DEPTH + 2]
    ssem = bufs_and_sems[4 * _DEPTH + 2:]
    wid = lax.axis_index("s") * _NC + lax.axis_index("c")
    base = wid * _B_PER_W
    pltpu.sync_copy(idx_hbm.at[pl.ds(base, _B_PER_W)], idx_v)

    def start_gather(c, p):
        sl = idx_v.at[pl.ds(c * _CHUNK, _CHUNK)]
        return (pltpu.async_copy(tl_hbm.at[sl], rows_l[p], gsem_l[p]),
                pltpu.async_copy(tr_hbm.at[sl], rows_r[p], gsem_r[p]))

    def expand_scale(p, w):
        def body(i, carry):
            wide[w][i, pl.ds(0, 16)] = rows_l[p][i, pl.ds(0, 16)] * SCALE
            wide[w][i, pl.ds(16, 16)] = rows_r[p][i, pl.ds(0, 16)] * SCALE
            return carry

        lax.fori_loop(0, _CHUNK, body, 0)

    gathers = [None] * _DEPTH
    stores = [None, None]
    for c in range(_N_CHUNKS + _DEPTH - 1):
        if c < _N_CHUNKS:
            p = c % _DEPTH
            gathers[p] = start_gather(c, p)
        d = c - (_DEPTH - 1)
        if d >= 0:
            q = d % _DEPTH
            w = d % 2
            gathers[q][0].wait()
            gathers[q][1].wait()
            if stores[w] is not None:
                stores[w].wait()
            expand_scale(q, w)
            stores[w] = pltpu.async_copy(
                wide[w], out_hbm.at[pl.ds(base + d * _CHUNK, _CHUNK)], ssem[w])
    for w in range(2):
        if stores[w] is not None:
            stores[w].wait()


def kernel(tokens, W):
    idx = tokens.reshape(-1).astype(jnp.int32)
    out128 = _embed_sc(idx, W[:, :16], W[:, 16:])
    return out128.reshape(4096, 200, 128)[:, :, :D_EMB]


# speedup vs baseline: 1.3406x; 1.3406x over previous
import functools

import jax
import jax.numpy as jnp
from jax import lax
from jax.experimental import pallas as pl
from jax.experimental.pallas import tpu as pltpu
from jax.experimental.pallas import tpu_sc as plsc

D_VOCAB = 1000000
D_EMB = 32
SCALE = float(D_EMB) ** 0.5

_NC = 2
_NS = 16
_NW = _NC * _NS

_B = 4096 * 200
_B_PER_W = _B // _NW
_CHUNK = 160
_N_CHUNKS = _B_PER_W // _CHUNK
_DEPTH = 4

_mesh = plsc.VectorSubcoreMesh(core_axis_name="c", subcore_axis_name="s")


@functools.partial(
    pl.kernel,
    mesh=_mesh,
    compiler_params=pltpu.CompilerParams(use_tc_tiling_on_sc=False),
    out_type=jax.ShapeDtypeStruct((_B, 64), jnp.float32),
    scratch_types=(
        [pltpu.VMEM((_B_PER_W,), jnp.int32)]
        + [pltpu.VMEM((_CHUNK, 32), jnp.float32) for _ in range(_DEPTH)]
        + [pltpu.VMEM((_CHUNK, 64), jnp.float32) for _ in range(2)]
        + [pltpu.SemaphoreType.DMA for _ in range(_DEPTH + 2)]
    ),
)
def _embed_sc(idx_hbm, table_hbm, out_hbm, idx_v, *bufs_and_sems):
    rows = bufs_and_sems[:_DEPTH]
    wide = bufs_and_sems[_DEPTH:_DEPTH + 2]
    gsem = bufs_and_sems[_DEPTH + 2:2 * _DEPTH + 2]
    ssem = bufs_and_sems[2 * _DEPTH + 2:]
    wid = lax.axis_index("s") * _NC + lax.axis_index("c")
    base = wid * _B_PER_W
    pltpu.sync_copy(idx_hbm.at[pl.ds(base, _B_PER_W)], idx_v)

    def start_gather(c, p):
        return pltpu.async_copy(
            table_hbm.at[idx_v.at[pl.ds(c * _CHUNK, _CHUNK)]], rows[p],
            gsem[p])

    def expand_scale(p, w):
        def body(i, carry):
            wide[w][i, pl.ds(0, 16)] = rows[p][i, pl.ds(0, 16)] * SCALE
            wide[w][i, pl.ds(16, 16)] = rows[p][i, pl.ds(16, 16)] * SCALE
            return carry

        lax.fori_loop(0, _CHUNK, body, 0)

    gathers = [None] * _DEPTH
    stores = [None, None]
    for c in range(_N_CHUNKS + _DEPTH - 1):
        if c < _N_CHUNKS:
            p = c % _DEPTH
            gathers[p] = start_gather(c, p)
        d = c - (_DEPTH - 1)
        if d >= 0:
            q = d % _DEPTH
            w = d % 2
            gathers[q].wait()
            if stores[w] is not None:
                stores[w].wait()
            expand_scale(q, w)
            stores[w] = pltpu.async_copy(
                wide[w], out_hbm.at[pl.ds(base + d * _CHUNK, _CHUNK)], ssem[w])
    for w in range(2):
        if stores[w] is not None:
            stores[w].wait()


def kernel(tokens, W):
    idx = tokens.reshape(-1).astype(jnp.int32)
    out64 = _embed_sc(idx, W)
    return out64.reshape(4096, 200, 64)[:, :, :D_EMB]


# speedup vs baseline: 1.7945x; 1.3386x over previous
import functools

import jax
import jax.numpy as jnp
from jax import lax
from jax.experimental import pallas as pl
from jax.experimental.pallas import tpu as pltpu
from jax.experimental.pallas import tpu_sc as plsc

D_VOCAB = 1000000
D_EMB = 32
SCALE = float(D_EMB) ** 0.5

_NC = 2
_NS = 16
_NW = _NC * _NS

_B = 4096 * 200
_B_PER_W = _B // _NW
_CHUNK = 160
_N_CHUNKS = _B_PER_W // _CHUNK
_DEPTH = 4

_mesh = plsc.VectorSubcoreMesh(core_axis_name="c", subcore_axis_name="s")


@functools.partial(
    pl.kernel,
    mesh=_mesh,
    compiler_params=pltpu.CompilerParams(use_tc_tiling_on_sc=False),
    out_type=jax.ShapeDtypeStruct((_B, 128), jnp.float32),
    scratch_types=(
        [pltpu.VMEM((_B_PER_W,), jnp.int32)]
        + [pltpu.VMEM((_CHUNK, 32), jnp.float32) for _ in range(_DEPTH)]
        + [pltpu.VMEM((_CHUNK, 128), jnp.float32) for _ in range(2)]
        + [pltpu.SemaphoreType.DMA for _ in range(_DEPTH + 2)]
    ),
)
def _embed_sc(idx_hbm, table_hbm, out_hbm, idx_v, *bufs_and_sems):
    rows = bufs_and_sems[:_DEPTH]
    wide = bufs_and_sems[_DEPTH:_DEPTH + 2]
    gsem = bufs_and_sems[_DEPTH + 2:2 * _DEPTH + 2]
    ssem = bufs_and_sems[2 * _DEPTH + 2:]
    wid = lax.axis_index("s") * _NC + lax.axis_index("c")
    base = wid * _B_PER_W
    pltpu.sync_copy(idx_hbm.at[pl.ds(base, _B_PER_W)], idx_v)

    def start_gather(c, p):
        return pltpu.async_copy(
            table_hbm.at[idx_v.at[pl.ds(c * _CHUNK, _CHUNK)]], rows[p],
            gsem[p])

    def expand_scale(p, w):
        def body(i, carry):
            wide[w][i, pl.ds(0, 16)] = rows[p][i, pl.ds(0, 16)] * SCALE
            wide[w][i, pl.ds(16, 16)] = rows[p][i, pl.ds(16, 16)] * SCALE
            return carry

        lax.fori_loop(0, _CHUNK, body, 0)

    gathers = [None] * _DEPTH
    stores = [None, None]
    for c in range(_N_CHUNKS + _DEPTH - 1):
        if c < _N_CHUNKS:
            p = c % _DEPTH
            gathers[p] = start_gather(c, p)
        d = c - (_DEPTH - 1)
        if d >= 0:
            q = d % _DEPTH
            w = d % 2
            gathers[q].wait()
            if stores[w] is not None:
                stores[w].wait()
            expand_scale(q, w)
            stores[w] = pltpu.async_copy(
                wide[w], out_hbm.at[pl.ds(base + d * _CHUNK, _CHUNK)], ssem[w])
    for w in range(2):
        if stores[w] is not None:
            stores[w].wait()


def kernel(tokens, W):
    idx = tokens.reshape(-1).astype(jnp.int32)
    out128 = _embed_sc(idx, W)
    return out128.reshape(4096, 200, 128)[:, :, :D_EMB]


# speedup vs baseline: 2.0402x; 1.1369x over previous
import functools

import jax
import jax.numpy as jnp
from jax import lax
from jax.experimental import pallas as pl
from jax.experimental.pallas import tpu as pltpu
from jax.experimental.pallas import tpu_sc as plsc

D_VOCAB = 1000000
D_EMB = 32
SCALE = float(D_EMB) ** 0.5

_NC = 2
_NS = 16
_NW = _NC * _NS

_B = 4096 * 200
_B_PER_W = _B // _NW
_CHUNK = 640
_N_CHUNKS = _B_PER_W // _CHUNK
_DEPTH = 4

_mesh = plsc.VectorSubcoreMesh(core_axis_name="c", subcore_axis_name="s")


@functools.partial(
    pl.kernel,
    mesh=_mesh,
    compiler_params=pltpu.CompilerParams(use_tc_tiling_on_sc=False),
    out_type=jax.ShapeDtypeStruct((_B, 128), jnp.float32),
    scratch_types=(
        [pltpu.VMEM((_B_PER_W,), jnp.int32)]
        + [pltpu.VMEM((_CHUNK, 32), jnp.float32) for _ in range(_DEPTH)]
        + [pltpu.SemaphoreType.DMA for _ in range(2 * _DEPTH)]
    ),
)
def _embed_sc(idx_hbm, table_hbm, out_hbm, idx_v, *bufs_and_sems):
    rows = bufs_and_sems[:_DEPTH]
    gsem = bufs_and_sems[_DEPTH:2 * _DEPTH]
    ssem = bufs_and_sems[2 * _DEPTH:]
    wid = lax.axis_index("s") * _NC + lax.axis_index("c")
    base = wid * _B_PER_W
    pltpu.sync_copy(idx_hbm.at[pl.ds(base, _B_PER_W)], idx_v)

    def start_gather(c, p):
        return pltpu.async_copy(
            table_hbm.at[idx_v.at[pl.ds(c * _CHUNK, _CHUNK)]], rows[p],
            gsem[p])

    def scale_buf(p):
        def body(i, carry):
            rows[p][i, pl.ds(0, 16)] = rows[p][i, pl.ds(0, 16)] * SCALE
            rows[p][i, pl.ds(16, 16)] = rows[p][i, pl.ds(16, 16)] * SCALE
            return carry

        lax.fori_loop(0, _CHUNK, body, 0)

    gathers = [None] * _DEPTH
    stores = [None] * _DEPTH
    for c in range(_N_CHUNKS + _DEPTH - 1):
        if c < _N_CHUNKS:
            p = c % _DEPTH
            if stores[p] is not None:
                stores[p].wait()
                stores[p] = None
            gathers[p] = start_gather(c, p)
        d = c - (_DEPTH - 1)
        if d >= 0:
            q = d % _DEPTH
            gathers[q].wait()
            scale_buf(q)
            stores[q] = pltpu.async_copy(
                rows[q],
                out_hbm.at[pl.ds(base + d * _CHUNK, _CHUNK), pl.ds(0, 32)],
                ssem[q])
    for q in range(_DEPTH):
        if stores[q] is not None:
            stores[q].wait()


def kernel(tokens, W):
    idx = tokens.reshape(-1).astype(jnp.int32)
    out128 = _embed_sc(idx, W)
    return out128.reshape(4096, 200, 128)[:, :, :D_EMB]
